# baseline (speedup 1.0000x reference)
import functools

import jax
import jax.numpy as jnp
from jax import lax
from jax.experimental import pallas as pl
from jax.experimental.pallas import tpu as pltpu
from jax.experimental.pallas import tpu_sc as plsc

N_TOTAL = 20000
N_SUB = 10000
E = 320000
D = 128
HID = 128
NUM_LATENT = 4
DK = 32
OUT_DIM = 64
T = 2048

ACCW = 144
NACC = 2056
SB = 2000
KCAP = SB + 16
EB = 48
NTILE = 16
EPT = E // NTILE
NBATCH = EPT // EB
TPT = T // NTILE


def _prep_body(x_ref, wfc_ref, bfc_ref, wz_ref, bz_ref, z_ref):
    t = jnp.dot(x_ref[...], wfc_ref[0], preferred_element_type=jnp.float32)
    t = t + bfc_ref[0]
    z = jnp.dot(t, wz_ref[0], preferred_element_type=jnp.float32)
    z_ref[0] = jnp.tanh(z + bz_ref[0])


def _make_ztables(feats, wfc, bfc, wz, bz):
    return pl.pallas_call(
        _prep_body,
        grid=(2, 2),
        in_specs=[
            pl.BlockSpec((N_SUB, D), lambda i, j: (i, 0)),
            pl.BlockSpec((1, D, HID), lambda i, j: (i, 0, 0)),
            pl.BlockSpec((1, 1, HID), lambda i, j: (i, 0, 0)),
            pl.BlockSpec((1, HID, HID), lambda i, j: (j, 0, 0)),
            pl.BlockSpec((1, 1, HID), lambda i, j: (j, 0, 0)),
        ],
        out_specs=pl.BlockSpec((1, N_SUB, HID), lambda i, j: (j, i, 0)),
        out_shape=jax.ShapeDtypeStruct((2, N_TOTAL, HID), jnp.float32),
    )(feats, wfc, bfc, wz, bz)


def _sc_body(zall, nidx, esrc, edst, tgt, mout,
             node_tab, m_tab, tgt_all, src_v, dst_v, gsrc_v, gdst_v, dslot_v,
             tslot_v, zsrc, zdst, rows, trow, mrow,
             ksrc, kdst, kslot, gsrc2, gdst2, dslot2, zsrc2, zdst2, rows2,
             hbm_dummy, acc, sem1, sem2, sem3, sem4, sem5, sem6, sem7, sem8):
    c = lax.axis_index("c")
    s = lax.axis_index("s")
    iota16 = lax.iota(jnp.int32, 16)
    zero16 = jnp.zeros((16,), jnp.float32)
    latm = [(iota16 == k).astype(jnp.float32) for k in range(NUM_LATENT)]

    def _zrow(i, carry):
        def _zcol(j, carry2):
            plsc.store_scatter(rows, [jnp.full((16,), i, jnp.int32),
                                      j * 16 + iota16], zero16)
            return carry2
        return lax.fori_loop(0, ACCW // 16, _zcol, carry)
    lax.fori_loop(0, EB, _zrow, 0)

    def _zacc(t, carry):
        chunk = s + 16 * t

        @pl.when(chunk < NACC // EB)
        def _():
            pltpu.sync_copy(rows,
                            acc.at[pl.ds(pl.multiple_of(chunk * EB, 8), EB)])

        @pl.when(chunk == NACC // EB)
        def _():
            pltpu.sync_copy(rows.at[pl.ds(0, NACC % EB)],
                            acc.at[pl.ds((NACC // EB) * EB, NACC % EB)])
        return carry
    lax.fori_loop(0, NACC // EB // 16 + 1, _zacc, 0)

    pltpu.sync_copy(nidx.at[pl.ds(pl.multiple_of(c * N_SUB, 8), N_SUB)],
                    node_tab)
    pltpu.sync_copy(tgt.at[pl.ds(pl.multiple_of(c * T, 8), T)], tgt_all)

    dump16 = jnp.full((16,), T, jnp.int32)

    def _minit(i, carry):
        m_tab[pl.ds(i * 16, 16)] = dump16
        return carry
    lax.fori_loop(0, N_SUB // 16, _minit, 0)

    def _mfill(g, carry):
        tv = tgt_all[pl.ds(g * 16, 16)]
        plsc.store_scatter(m_tab, [tv], iota16 + g * 16)
        return carry
    lax.fori_loop(0, T // 16, _mfill, 0)

    plsc.subcore_barrier()

    zofs = c * N_TOTAL
    ebase = c * E + s * EPT

    def _kinit(i, carry):
        z16 = jnp.zeros((16,), jnp.int32)
        ksrc[pl.ds(i * 16, 16)] = z16
        kdst[pl.ds(i * 16, 16)] = z16
        return carry
    lax.fori_loop(0, KCAP // 16, _kinit, 0)

    def _super(sb, carry):
        def _ks(i, carry2):
            kslot[pl.ds(i * 16, 16)] = dump16
            return carry2
        lax.fori_loop(0, KCAP // 16, _ks, 0)

        sbase = pl.multiple_of(ebase + sb * SB, 8)

        cpa = pltpu.async_copy(esrc.at[pl.ds(sbase, SB)], src_v, sem1)
        cpb = pltpu.async_copy(edst.at[pl.ds(sbase, SB)], dst_v, sem2)
        cpa.wait()
        cpb.wait()

        def _cgrp(g, cnt2):
            sv = src_v[pl.ds(g * 16, 16)]
            dv = dst_v[pl.ds(g * 16, 16)]
            slot16 = plsc.load_gather(m_tab, [dv])
            mask = slot16 != dump16
            cs = plsc.cumsum(mask.astype(jnp.int32))
            pos = cnt2 + cs - 1
            plsc.store_scatter(ksrc, [pos], sv, mask=mask)
            plsc.store_scatter(kdst, [pos], dv, mask=mask)
            plsc.store_scatter(kslot, [pos], slot16, mask=mask)
            return cnt2 + jnp.max(cs)
        nk = lax.fori_loop(0, SB // 16, _cgrp, jnp.int32(0))
        nb = (nk + EB - 1) // EB

        def _fire(b, gsrc_p, gdst_p, dslot_p, zsrc_p, zdst_p, sga, sgb,
                  rows_p, ssem_p):
            @pl.when(b >= 2)
            def _():
                pltpu.make_async_copy(hbm_dummy, rows_p, ssem_p).wait()

            k0 = b * EB

            def _comp(g, carry3):
                k16 = k0 + g * 16 + iota16
                sv = plsc.load_gather(ksrc, [k16])
                gsrc_p[pl.ds(g * 16, 16)] = (plsc.load_gather(node_tab, [sv])
                                             + zofs)
                dv = plsc.load_gather(kdst, [k16])
                gdst_p[pl.ds(g * 16, 16)] = (plsc.load_gather(node_tab, [dv])
                                             + zofs)
                dslot_p[pl.ds(g * 16, 16)] = plsc.load_gather(kslot, [k16])
                return carry3
            lax.fori_loop(0, EB // 16, _comp, 0)
            pltpu.async_copy(zall.at[gsrc_p], zsrc_p, sga)
            pltpu.async_copy(zall.at[gdst_p], zdst_p, sgb)

        def _process(b, gsrc_p, gdst_p, dslot_p, zsrc_p, zdst_p, sga, sgb,
                     rows_p, ssem_p):
            pltpu.make_async_copy(zall.at[pl.ds(0, EB)], zsrc_p, sga).wait()
            pltpu.make_async_copy(zall.at[pl.ds(0, EB)], zdst_p, sgb).wait()

            def _edge(r2, carry3):
                for dr in range(2):
                    r = r2 * 2 + dr
                    rfull = jnp.full((16,), r, jnp.int32)
                    av = [plsc.load_gather(zsrc_p, [rfull, j * 16 + iota16])
                          for j in range(HID // 16)]
                    bv = [plsc.load_gather(zdst_p, [rfull, j * 16 + iota16])
                          for j in range(HID // 16)]
                    dvec = jnp.zeros((16,), jnp.float32)
                    for k in range(NUM_LATENT):
                        p = (av[2 * k] * bv[2 * k]
                             + av[2 * k + 1] * bv[2 * k + 1])
                        s = jnp.sum(p)
                        e = jnp.maximum(s, s * 0.2)
                        eev = jnp.exp(jnp.full((16,), e, jnp.float32))
                        plsc.store_scatter(rows_p, [rfull, k * DK + iota16],
                                           av[2 * k] * eev)
                        plsc.store_scatter(rows_p,
                                           [rfull, k * DK + 16 + iota16],
                                           av[2 * k + 1] * eev)
                        dvec = dvec + eev * latm[k]
                    plsc.store_scatter(rows_p, [rfull, 128 + iota16], dvec)
                return carry3
            lax.fori_loop(0, EB // 2, _edge, 0)

            pltpu.async_copy(rows_p, acc.at[dslot_p], ssem_p, add=True)

        p0 = (gsrc_v, gdst_v, dslot_v, zsrc, zdst, sem3, sem4, rows, sem7)
        p1 = (gsrc2, gdst2, dslot2, zsrc2, zdst2, sem5, sem6, rows2, sem8)

        @pl.when(nb > 0)
        def _():
            _fire(0, *p0)

        def _pairs(i, carry2):
            b0 = 2 * i
            b1 = b0 + 1

            @pl.when(b1 < nb)
            def _():
                _fire(b1, *p1)

            @pl.when(b0 < nb)
            def _():
                _process(b0, *p0)

            @pl.when(b1 + 1 < nb)
            def _():
                _fire(b1 + 1, *p0)

            @pl.when(b1 < nb)
            def _():
                _process(b1, *p1)
            return carry2
        lax.fori_loop(0, (nb + 1) // 2, _pairs, 0)

        @pl.when(((nb >= 1) & ((nb - 1) % 2 == 0)) | (nb >= 2))
        def _():
            pltpu.make_async_copy(hbm_dummy, rows, sem7).wait()

        @pl.when((nb >= 2) | ((nb >= 1) & ((nb - 1) % 2 == 1)))
        def _():
            pltpu.make_async_copy(hbm_dummy, rows2, sem8).wait()
        return carry
    lax.fori_loop(0, EPT // SB, _super, 0)

    plsc.subcore_barrier()

    tb = pl.multiple_of(s * TPT, 8)

    def _tslot(g, carry):
        tv = tgt_all[pl.ds(tb + g * 16, 16)]
        tslot_v[pl.ds(g * 16, 16)] = plsc.load_gather(m_tab, [tv])
        return carry
    lax.fori_loop(0, TPT // 16, _tslot, 0)

    pltpu.async_copy(acc.at[tslot_v], trow, sem1).wait()

    def _nrm(i, carry):
        ifull = jnp.full((16,), i, jnp.int32)
        for k in range(NUM_LATENT):
            dk = plsc.load_gather(trow, [ifull,
                                         jnp.full((16,), 128 + k, jnp.int32)])
            dk = dk + 1e-9
            for j2 in range(2):
                off = k * DK + j2 * 16
                v = plsc.load_gather(trow, [ifull, off + iota16]) / dk
                plsc.store_scatter(mrow, [ifull, off + iota16], v)
        return carry
    lax.fori_loop(0, TPT, _nrm, 0)

    pltpu.sync_copy(mrow, mout.at[c, pl.ds(tb, TPT)])


def _sc_aggregate(zall, nidx, esrc, edst, tgt):
    mesh = plsc.VectorSubcoreMesh(core_axis_name="c", subcore_axis_name="s")
    return pl.kernel(
        _sc_body,
        out_type=jax.ShapeDtypeStruct((2, T, HID), jnp.float32),
        mesh=mesh,
        compiler_params=pltpu.CompilerParams(use_tc_tiling_on_sc=False,
                                             needs_layout_passes=False),
        scratch_types=[
            pltpu.VMEM((N_SUB,), jnp.int32),
            pltpu.VMEM((N_SUB,), jnp.int32),
            pltpu.VMEM((T,), jnp.int32),
            pltpu.VMEM((SB,), jnp.int32),
            pltpu.VMEM((SB,), jnp.int32),
            pltpu.VMEM((EB,), jnp.int32),
            pltpu.VMEM((EB,), jnp.int32),
            pltpu.VMEM((EB,), jnp.int32),
            pltpu.VMEM((TPT,), jnp.int32),
            pltpu.VMEM((EB, HID), jnp.float32),
            pltpu.VMEM((EB, HID), jnp.float32),
            pltpu.VMEM((EB, ACCW), jnp.float32),
            pltpu.VMEM((TPT, ACCW), jnp.float32),
            pltpu.VMEM((TPT, HID), jnp.float32),
            pltpu.VMEM((KCAP,), jnp.int32),
            pltpu.VMEM((KCAP,), jnp.int32),
            pltpu.VMEM((KCAP,), jnp.int32),
            pltpu.VMEM((EB,), jnp.int32),
            pltpu.VMEM((EB,), jnp.int32),
            pltpu.VMEM((EB,), jnp.int32),
            pltpu.VMEM((EB, HID), jnp.float32),
            pltpu.VMEM((EB, HID), jnp.float32),
            pltpu.VMEM((EB, ACCW), jnp.float32),
            pltpu.HBM((EB, ACCW), jnp.float32),
            pltpu.VMEM_SHARED((NACC, ACCW), jnp.float32),
        ] + [pltpu.SemaphoreType.DMA] * 8,
    )(zall, nidx, esrc, edst, tgt)


def _out_body(m_ref, w_ref, b_ref, o_ref):
    o = jnp.dot(m_ref[0], w_ref[0], preferred_element_type=jnp.float32)
    o_ref[0] = o + b_ref[0]


def _project_out(mout, w, b):
    return pl.pallas_call(
        _out_body,
        grid=(2,),
        in_specs=[
            pl.BlockSpec((1, T, HID), lambda j: (j, 0, 0)),
            pl.BlockSpec((1, HID, OUT_DIM), lambda j: (j, 0, 0)),
            pl.BlockSpec((1, 1, OUT_DIM), lambda j: (j, 0, 0)),
        ],
        out_specs=pl.BlockSpec((1, T, OUT_DIM), lambda j: (j, 0, 0)),
        out_shape=jax.ShapeDtypeStruct((2, T, OUT_DIM), jnp.float32),
    )(mout, w, b)


def kernel(feat0, feat1, type_mask, node_idx_gene, node_idx_dis,
           edge_index_gene, edge_index_dis, target_idx_gene, target_idx_dis,
           fc_type_W, fc_type_b, gene_Wf, gene_bf, gene_fc1_W, gene_fc1_b,
           gene_fc2_W, gene_fcout_W, gene_fcout_b, dis_Wf, dis_bf, dis_fc1_W,
           dis_fc1_b, dis_fc2_W, dis_fcout_W, dis_fcout_b):
    feats = jnp.concatenate([feat0, feat1], axis=0)
    wz = jnp.stack([
        jnp.transpose(gene_Wf, (1, 0, 2)).reshape(HID, HID),
        jnp.transpose(dis_Wf, (1, 0, 2)).reshape(HID, HID),
    ])
    bz = jnp.stack([gene_bf.reshape(1, HID), dis_bf.reshape(1, HID)])

    zall = _make_ztables(feats, fc_type_W, fc_type_b.reshape(2, 1, HID), wz, bz)
    zflat = zall.reshape(2 * N_TOTAL, HID)

    nidx = jnp.concatenate([node_idx_gene, node_idx_dis])
    esrc = jnp.concatenate([edge_index_gene[0], edge_index_dis[0]])
    edst = jnp.concatenate([edge_index_gene[1], edge_index_dis[1]])
    tgt = jnp.concatenate([target_idx_gene, target_idx_dis])

    mout = _sc_aggregate(zflat, nidx, esrc, edst, tgt)

    wout = jnp.stack([gene_fcout_W, dis_fcout_W])
    bout = jnp.stack([gene_fcout_b.reshape(1, OUT_DIM),
                      dis_fcout_b.reshape(1, OUT_DIM)])
    logits = _project_out(mout, wout, bout)
    return (logits[0], logits[1])


# speedup vs baseline: 1.0037x; 1.0037x over previous
import functools

import jax
import jax.numpy as jnp
from jax import lax
from jax.experimental import pallas as pl
from jax.experimental.pallas import tpu as pltpu
from jax.experimental.pallas import tpu_sc as plsc

N_TOTAL = 20000
N_SUB = 10000
E = 320000
D = 128
HID = 128
NUM_LATENT = 4
DK = 32
OUT_DIM = 64
T = 2048

ACCW = 144
NACC = 2056
SB = 2000
KCAP = SB + 16
EB = 48
NTILE = 16
EPT = E // NTILE
NBATCH = EPT // EB
TPT = T // NTILE


def _prep_body(x_ref, wfc_ref, bfc_ref, wz_ref, bz_ref, z_ref):
    t = jnp.dot(x_ref[...], wfc_ref[0], preferred_element_type=jnp.float32)
    t = t + bfc_ref[0]
    z = jnp.dot(t, wz_ref[0], preferred_element_type=jnp.float32)
    z_ref[0] = jnp.tanh(z + bz_ref[0])


def _make_ztables(feats, wfc, bfc, wz, bz):
    return pl.pallas_call(
        _prep_body,
        grid=(2, 2),
        in_specs=[
            pl.BlockSpec((N_SUB, D), lambda i, j: (i, 0)),
            pl.BlockSpec((1, D, HID), lambda i, j: (i, 0, 0)),
            pl.BlockSpec((1, 1, HID), lambda i, j: (i, 0, 0)),
            pl.BlockSpec((1, HID, HID), lambda i, j: (j, 0, 0)),
            pl.BlockSpec((1, 1, HID), lambda i, j: (j, 0, 0)),
        ],
        out_specs=pl.BlockSpec((1, N_SUB, HID), lambda i, j: (j, i, 0)),
        out_shape=jax.ShapeDtypeStruct((2, N_TOTAL, HID), jnp.float32),
    )(feats, wfc, bfc, wz, bz)


def _sc_body(zall, nidx, esrc, edst, tgt, mout,
             node_tab, m_tab, tgt_all, src_v, dst_v, gsrc_v, gdst_v, dslot_v,
             tslot_v, zsrc, zdst, rows, trow, mrow,
             ksrc, kdst, kslot, gsrc2, gdst2, dslot2, zsrc2, zdst2, rows2,
             hbm_dummy, acc, sem1, sem2, sem3, sem4, sem5, sem6, sem7, sem8):
    c = lax.axis_index("c")
    s = lax.axis_index("s")
    iota16 = lax.iota(jnp.int32, 16)
    zero16 = jnp.zeros((16,), jnp.float32)
    latm = [(iota16 == k).astype(jnp.float32) for k in range(NUM_LATENT)]

    def _zrow(i, carry):
        def _zcol(j, carry2):
            plsc.store_scatter(rows, [jnp.full((16,), i, jnp.int32),
                                      j * 16 + iota16], zero16)
            return carry2
        return lax.fori_loop(0, ACCW // 16, _zcol, carry)
    lax.fori_loop(0, EB, _zrow, 0)

    def _zacc(t, carry):
        chunk = s + 16 * t

        @pl.when(chunk < NACC // EB)
        def _():
            pltpu.sync_copy(rows,
                            acc.at[pl.ds(pl.multiple_of(chunk * EB, 8), EB)])

        @pl.when(chunk == NACC // EB)
        def _():
            pltpu.sync_copy(rows.at[pl.ds(0, NACC % EB)],
                            acc.at[pl.ds((NACC // EB) * EB, NACC % EB)])
        return carry
    lax.fori_loop(0, NACC // EB // 16 + 1, _zacc, 0)

    pltpu.sync_copy(nidx.at[pl.ds(pl.multiple_of(c * N_SUB, 8), N_SUB)],
                    node_tab)
    pltpu.sync_copy(tgt.at[pl.ds(pl.multiple_of(c * T, 8), T)], tgt_all)

    dump16 = jnp.full((16,), T, jnp.int32)

    def _minit(i, carry):
        m_tab[pl.ds(i * 16, 16)] = dump16
        return carry
    lax.fori_loop(0, N_SUB // 16, _minit, 0)

    def _mfill(g, carry):
        tv = tgt_all[pl.ds(g * 16, 16)]
        plsc.store_scatter(m_tab, [tv], iota16 + g * 16)
        return carry
    lax.fori_loop(0, T // 16, _mfill, 0)

    plsc.subcore_barrier()

    zofs = c * N_TOTAL
    ebase = c * E + s * EPT

    def _kinit(i, carry):
        z16 = jnp.zeros((16,), jnp.int32)
        ksrc[pl.ds(i * 16, 16)] = z16
        kdst[pl.ds(i * 16, 16)] = z16
        return carry
    lax.fori_loop(0, KCAP // 16, _kinit, 0)

    def _super(sb, carry):
        def _ks(i, carry2):
            kslot[pl.ds(i * 16, 16)] = dump16
            return carry2
        lax.fori_loop(0, KCAP // 16, _ks, 0)

        sbase = pl.multiple_of(ebase + sb * SB, 8)

        cpa = pltpu.async_copy(esrc.at[pl.ds(sbase, SB)], src_v, sem1)
        cpb = pltpu.async_copy(edst.at[pl.ds(sbase, SB)], dst_v, sem2)
        cpa.wait()
        cpb.wait()

        def _cgrp(g, cnt2):
            sv = src_v[pl.ds(g * 16, 16)]
            dv = dst_v[pl.ds(g * 16, 16)]
            slot16 = plsc.load_gather(m_tab, [dv])
            mask = slot16 != dump16
            cs = plsc.cumsum(mask.astype(jnp.int32))
            pos = cnt2 + cs - 1
            plsc.store_scatter(ksrc, [pos], sv, mask=mask)
            plsc.store_scatter(kdst, [pos], dv, mask=mask)
            plsc.store_scatter(kslot, [pos], slot16, mask=mask)
            return cnt2 + jnp.max(cs)
        nk = lax.fori_loop(0, SB // 16, _cgrp, jnp.int32(0))
        nb = (nk + EB - 1) // EB

        def _fire(b, gsrc_p, gdst_p, dslot_p, zsrc_p, zdst_p, sga, sgb,
                  rows_p, ssem_p):
            @pl.when(b >= 2)
            def _():
                pltpu.make_async_copy(hbm_dummy, rows_p, ssem_p).wait()

            k0 = b * EB

            def _comp(g, carry3):
                k16 = k0 + g * 16 + iota16
                sv = plsc.load_gather(ksrc, [k16])
                gsrc_p[pl.ds(g * 16, 16)] = (plsc.load_gather(node_tab, [sv])
                                             + zofs)
                dv = plsc.load_gather(kdst, [k16])
                gdst_p[pl.ds(g * 16, 16)] = (plsc.load_gather(node_tab, [dv])
                                             + zofs)
                dslot_p[pl.ds(g * 16, 16)] = plsc.load_gather(kslot, [k16])
                return carry3
            lax.fori_loop(0, EB // 16, _comp, 0)
            for g in range(EB // 16):
                sl = pl.ds(g * 16, 16)
                pltpu.async_copy(zall.at[gsrc_p.at[sl]], zsrc_p.at[sl], sga)
                pltpu.async_copy(zall.at[gdst_p.at[sl]], zdst_p.at[sl], sgb)

        def _process(b, gsrc_p, gdst_p, dslot_p, zsrc_p, zdst_p, sga, sgb,
                     rows_p, ssem_p):
            for g in range(EB // 16):
                sl = pl.ds(g * 16, 16)
                pltpu.make_async_copy(zall.at[pl.ds(0, 16)],
                                      zsrc_p.at[sl], sga).wait()
                pltpu.make_async_copy(zall.at[pl.ds(0, 16)],
                                      zdst_p.at[sl], sgb).wait()

            def _edge(r2, carry3):
                for dr in range(1):
                    r = r2 + dr
                    rfull = jnp.full((16,), r, jnp.int32)
                    av = [plsc.load_gather(zsrc_p, [rfull, j * 16 + iota16])
                          for j in range(HID // 16)]
                    bv = [plsc.load_gather(zdst_p, [rfull, j * 16 + iota16])
                          for j in range(HID // 16)]
                    dvec = jnp.zeros((16,), jnp.float32)
                    for k in range(NUM_LATENT):
                        p = (av[2 * k] * bv[2 * k]
                             + av[2 * k + 1] * bv[2 * k + 1])
                        s = jnp.sum(p)
                        e = jnp.maximum(s, s * 0.2)
                        eev = jnp.exp(jnp.full((16,), e, jnp.float32))
                        plsc.store_scatter(rows_p, [rfull, k * DK + iota16],
                                           av[2 * k] * eev)
                        plsc.store_scatter(rows_p,
                                           [rfull, k * DK + 16 + iota16],
                                           av[2 * k + 1] * eev)
                        dvec = dvec + eev * latm[k]
                    plsc.store_scatter(rows_p, [rfull, 128 + iota16], dvec)
                return carry3
            lax.fori_loop(0, EB, _edge, 0)

            pltpu.async_copy(rows_p, acc.at[dslot_p], ssem_p, add=True)

        p0 = (gsrc_v, gdst_v, dslot_v, zsrc, zdst, sem3, sem4, rows, sem7)
        p1 = (gsrc2, gdst2, dslot2, zsrc2, zdst2, sem5, sem6, rows2, sem8)

        @pl.when(nb > 0)
        def _():
            _fire(0, *p0)

        def _pairs(i, carry2):
            b0 = 2 * i
            b1 = b0 + 1

            @pl.when(b1 < nb)
            def _():
                _fire(b1, *p1)

            @pl.when(b0 < nb)
            def _():
                _process(b0, *p0)

            @pl.when(b1 + 1 < nb)
            def _():
                _fire(b1 + 1, *p0)

            @pl.when(b1 < nb)
            def _():
                _process(b1, *p1)
            return carry2
        lax.fori_loop(0, (nb + 1) // 2, _pairs, 0)

        @pl.when(((nb >= 1) & ((nb - 1) % 2 == 0)) | (nb >= 2))
        def _():
            pltpu.make_async_copy(hbm_dummy, rows, sem7).wait()

        @pl.when((nb >= 2) | ((nb >= 1) & ((nb - 1) % 2 == 1)))
        def _():
            pltpu.make_async_copy(hbm_dummy, rows2, sem8).wait()
        return carry
    lax.fori_loop(0, EPT // SB, _super, 0)

    plsc.subcore_barrier()

    tb = pl.multiple_of(s * TPT, 8)

    def _tslot(g, carry):
        tv = tgt_all[pl.ds(tb + g * 16, 16)]
        tslot_v[pl.ds(g * 16, 16)] = plsc.load_gather(m_tab, [tv])
        return carry
    lax.fori_loop(0, TPT // 16, _tslot, 0)

    pltpu.async_copy(acc.at[tslot_v], trow, sem1).wait()

    def _nrm(i, carry):
        ifull = jnp.full((16,), i, jnp.int32)
        for k in range(NUM_LATENT):
            dk = plsc.load_gather(trow, [ifull,
                                         jnp.full((16,), 128 + k, jnp.int32)])
            dk = dk + 1e-9
            for j2 in range(2):
                off = k * DK + j2 * 16
                v = plsc.load_gather(trow, [ifull, off + iota16]) / dk
                plsc.store_scatter(mrow, [ifull, off + iota16], v)
        return carry
    lax.fori_loop(0, TPT, _nrm, 0)

    pltpu.sync_copy(mrow, mout.at[c, pl.ds(tb, TPT)])


def _sc_aggregate(zall, nidx, esrc, edst, tgt):
    mesh = plsc.VectorSubcoreMesh(core_axis_name="c", subcore_axis_name="s")
    return pl.kernel(
        _sc_body,
        out_type=jax.ShapeDtypeStruct((2, T, HID), jnp.float32),
        mesh=mesh,
        compiler_params=pltpu.CompilerParams(use_tc_tiling_on_sc=False,
                                             needs_layout_passes=False),
        scratch_types=[
            pltpu.VMEM((N_SUB,), jnp.int32),
            pltpu.VMEM((N_SUB,), jnp.int32),
            pltpu.VMEM((T,), jnp.int32),
            pltpu.VMEM((SB,), jnp.int32),
            pltpu.VMEM((SB,), jnp.int32),
            pltpu.VMEM((EB,), jnp.int32),
            pltpu.VMEM((EB,), jnp.int32),
            pltpu.VMEM((EB,), jnp.int32),
            pltpu.VMEM((TPT,), jnp.int32),
            pltpu.VMEM((EB, HID), jnp.float32),
            pltpu.VMEM((EB, HID), jnp.float32),
            pltpu.VMEM((EB, ACCW), jnp.float32),
            pltpu.VMEM((TPT, ACCW), jnp.float32),
            pltpu.VMEM((TPT, HID), jnp.float32),
            pltpu.VMEM((KCAP,), jnp.int32),
            pltpu.VMEM((KCAP,), jnp.int32),
            pltpu.VMEM((KCAP,), jnp.int32),
            pltpu.VMEM((EB,), jnp.int32),
            pltpu.VMEM((EB,), jnp.int32),
            pltpu.VMEM((EB,), jnp.int32),
            pltpu.VMEM((EB, HID), jnp.float32),
            pltpu.VMEM((EB, HID), jnp.float32),
            pltpu.VMEM((EB, ACCW), jnp.float32),
            pltpu.HBM((EB, ACCW), jnp.float32),
            pltpu.VMEM_SHARED((NACC, ACCW), jnp.float32),
        ] + [pltpu.SemaphoreType.DMA] * 8,
    )(zall, nidx, esrc, edst, tgt)


def _out_body(m_ref, w_ref, b_ref, o_ref):
    o = jnp.dot(m_ref[0], w_ref[0], preferred_element_type=jnp.float32)
    o_ref[0] = o + b_ref[0]


def _project_out(mout, w, b):
    return pl.pallas_call(
        _out_body,
        grid=(2,),
        in_specs=[
            pl.BlockSpec((1, T, HID), lambda j: (j, 0, 0)),
            pl.BlockSpec((1, HID, OUT_DIM), lambda j: (j, 0, 0)),
            pl.BlockSpec((1, 1, OUT_DIM), lambda j: (j, 0, 0)),
        ],
        out_specs=pl.BlockSpec((1, T, OUT_DIM), lambda j: (j, 0, 0)),
        out_shape=jax.ShapeDtypeStruct((2, T, OUT_DIM), jnp.float32),
    )(mout, w, b)


def kernel(feat0, feat1, type_mask, node_idx_gene, node_idx_dis,
           edge_index_gene, edge_index_dis, target_idx_gene, target_idx_dis,
           fc_type_W, fc_type_b, gene_Wf, gene_bf, gene_fc1_W, gene_fc1_b,
           gene_fc2_W, gene_fcout_W, gene_fcout_b, dis_Wf, dis_bf, dis_fc1_W,
           dis_fc1_b, dis_fc2_W, dis_fcout_W, dis_fcout_b):
    feats = jnp.concatenate([feat0, feat1], axis=0)
    wz = jnp.stack([
        jnp.transpose(gene_Wf, (1, 0, 2)).reshape(HID, HID),
        jnp.transpose(dis_Wf, (1, 0, 2)).reshape(HID, HID),
    ])
    bz = jnp.stack([gene_bf.reshape(1, HID), dis_bf.reshape(1, HID)])

    zall = _make_ztables(feats, fc_type_W, fc_type_b.reshape(2, 1, HID), wz, bz)
    zflat = zall.reshape(2 * N_TOTAL, HID)

    nidx = jnp.concatenate([node_idx_gene, node_idx_dis])
    esrc = jnp.concatenate([edge_index_gene[0], edge_index_dis[0]])
    edst = jnp.concatenate([edge_index_gene[1], edge_index_dis[1]])
    tgt = jnp.concatenate([target_idx_gene, target_idx_dis])

    mout = _sc_aggregate(zflat, nidx, esrc, edst, tgt)

    wout = jnp.stack([gene_fcout_W, dis_fcout_W])
    bout = jnp.stack([gene_fcout_b.reshape(1, OUT_DIM),
                      dis_fcout_b.reshape(1, OUT_DIM)])
    logits = _project_out(mout, wout, bout)
    return (logits[0], logits[1])


# speedup vs baseline: 1.1004x; 1.0963x over previous
import functools

import jax
import jax.numpy as jnp
from jax import lax
from jax.experimental import pallas as pl
from jax.experimental.pallas import tpu as pltpu
from jax.experimental.pallas import tpu_sc as plsc

N_TOTAL = 20000
N_SUB = 10000
E = 320000
D = 128
HID = 128
NUM_LATENT = 4
DK = 32
OUT_DIM = 64
T = 2048

ACCW = 144
NACC = 2056
SB = 2000
KCAP = SB + 16
EB = 48
NTILE = 16
EPT = E // NTILE
NBATCH = EPT // EB
TPT = T // NTILE


def _prep_body(x_ref, wfc_ref, bfc_ref, wz_ref, bz_ref, z_ref):
    t = jnp.dot(x_ref[...], wfc_ref[0], preferred_element_type=jnp.float32)
    t = t + bfc_ref[0]
    z = jnp.dot(t, wz_ref[0], preferred_element_type=jnp.float32)
    z_ref[0] = jnp.tanh(z + bz_ref[0])


def _make_ztables(feats, wfc, bfc, wz, bz):
    return pl.pallas_call(
        _prep_body,
        grid=(2, 2),
        in_specs=[
            pl.BlockSpec((N_SUB, D), lambda i, j: (i, 0)),
            pl.BlockSpec((1, D, HID), lambda i, j: (i, 0, 0)),
            pl.BlockSpec((1, 1, HID), lambda i, j: (i, 0, 0)),
            pl.BlockSpec((1, HID, HID), lambda i, j: (j, 0, 0)),
            pl.BlockSpec((1, 1, HID), lambda i, j: (j, 0, 0)),
        ],
        out_specs=pl.BlockSpec((1, N_SUB, HID), lambda i, j: (j, i, 0)),
        out_shape=jax.ShapeDtypeStruct((2, N_TOTAL, HID), jnp.float32),
    )(feats, wfc, bfc, wz, bz)


def _sc_body(zall, nidx, esrc, edst, tgt, mout,
             node_tab, m_tab, tgt_all, src_v, dst_v, gsrc_v, gdst_v, dslot_v,
             tslot_v, zsrc, zdst, rows, trow, mrow,
             ksrc, kdst, kslot, gsrc2, gdst2, dslot2, zsrc2, zdst2, rows2,
             hbm_dummy, acc, zt,
             sem1, sem2, sem3, sem4, sem5, sem6, sem7, sem8):
    c = lax.axis_index("c")
    s = lax.axis_index("s")
    iota16 = lax.iota(jnp.int32, 16)
    zero16 = jnp.zeros((16,), jnp.float32)
    latm = [(iota16 == k).astype(jnp.float32) for k in range(NUM_LATENT)]

    def _zrow(i, carry):
        def _zcol(j, carry2):
            plsc.store_scatter(rows, [jnp.full((16,), i, jnp.int32),
                                      j * 16 + iota16], zero16)
            return carry2
        return lax.fori_loop(0, ACCW // 16, _zcol, carry)
    lax.fori_loop(0, EB, _zrow, 0)

    def _zacc(t, carry):
        chunk = s + 16 * t

        @pl.when(chunk < NACC // EB)
        def _():
            pltpu.sync_copy(rows,
                            acc.at[pl.ds(pl.multiple_of(chunk * EB, 8), EB)])

        @pl.when(chunk == NACC // EB)
        def _():
            pltpu.sync_copy(rows.at[pl.ds(0, NACC % EB)],
                            acc.at[pl.ds((NACC // EB) * EB, NACC % EB)])
        return carry
    lax.fori_loop(0, NACC // EB // 16 + 1, _zacc, 0)

    pltpu.sync_copy(nidx.at[pl.ds(pl.multiple_of(c * N_SUB, 8), N_SUB)],
                    node_tab)
    pltpu.sync_copy(tgt.at[pl.ds(pl.multiple_of(c * T, 8), T)], tgt_all)

    dump16 = jnp.full((16,), T, jnp.int32)

    def _minit(i, carry):
        m_tab[pl.ds(i * 16, 16)] = dump16
        return carry
    lax.fori_loop(0, N_SUB // 16, _minit, 0)

    def _mfill(g, carry):
        tv = tgt_all[pl.ds(g * 16, 16)]
        plsc.store_scatter(m_tab, [tv], iota16 + g * 16)
        return carry
    lax.fori_loop(0, T // 16, _mfill, 0)

    tb0 = s * TPT

    def _stage(g, carry):
        tv = tgt_all[pl.ds(tb0 + g * 16, 16)]
        gsrc_v[pl.ds(0, 16)] = plsc.load_gather(node_tab, [tv]) + c * N_TOTAL
        pltpu.async_copy(zall.at[gsrc_v.at[pl.ds(0, 16)]],
                         zsrc.at[pl.ds(0, 16)], sem1).wait()
        pltpu.sync_copy(zsrc.at[pl.ds(0, 16)],
                        zt.at[pl.ds(pl.multiple_of(tb0 + g * 16, 8), 16)])
        return carry
    lax.fori_loop(0, TPT // 16, _stage, 0)

    plsc.subcore_barrier()

    zofs = c * N_TOTAL
    ebase = c * E + s * EPT

    def _kinit(i, carry):
        z16 = jnp.zeros((16,), jnp.int32)
        ksrc[pl.ds(i * 16, 16)] = z16
        kdst[pl.ds(i * 16, 16)] = z16
        return carry
    lax.fori_loop(0, KCAP // 16, _kinit, 0)

    def _super(sb, carry):
        def _ks(i, carry2):
            kslot[pl.ds(i * 16, 16)] = dump16
            return carry2
        lax.fori_loop(0, KCAP // 16, _ks, 0)

        sbase = pl.multiple_of(ebase + sb * SB, 8)

        cpa = pltpu.async_copy(esrc.at[pl.ds(sbase, SB)], src_v, sem1)
        cpb = pltpu.async_copy(edst.at[pl.ds(sbase, SB)], dst_v, sem2)
        cpa.wait()
        cpb.wait()

        def _cgrp(g, cnt2):
            sv = src_v[pl.ds(g * 16, 16)]
            dv = dst_v[pl.ds(g * 16, 16)]
            slot16 = plsc.load_gather(m_tab, [dv])
            mask = slot16 != dump16
            cs = plsc.cumsum(mask.astype(jnp.int32))
            pos = cnt2 + cs - 1
            plsc.store_scatter(ksrc, [pos], sv, mask=mask)
            plsc.store_scatter(kdst, [pos], dv, mask=mask)
            plsc.store_scatter(kslot, [pos], slot16, mask=mask)
            return cnt2 + jnp.max(cs)
        nk = lax.fori_loop(0, SB // 16, _cgrp, jnp.int32(0))
        nb = (nk + EB - 1) // EB

        def _fire(b, gsrc_p, gdst_p, dslot_p, zsrc_p, zdst_p, sga, sgb,
                  rows_p, ssem_p):
            @pl.when(b >= 2)
            def _():
                pltpu.make_async_copy(hbm_dummy, rows_p, ssem_p).wait()

            k0 = b * EB

            def _comp(g, carry3):
                k16 = k0 + g * 16 + iota16
                sv = plsc.load_gather(ksrc, [k16])
                gsrc_p[pl.ds(g * 16, 16)] = (plsc.load_gather(node_tab, [sv])
                                             + zofs)
                dslot_p[pl.ds(g * 16, 16)] = plsc.load_gather(kslot, [k16])
                return carry3
            lax.fori_loop(0, EB // 16, _comp, 0)
            for g in range(EB // 16):
                sl = pl.ds(g * 16, 16)
                pltpu.async_copy(zall.at[gsrc_p.at[sl]], zsrc_p.at[sl], sga)
                pltpu.async_copy(zt.at[dslot_p.at[sl]], zdst_p.at[sl], sgb)

        def _process(b, gsrc_p, gdst_p, dslot_p, zsrc_p, zdst_p, sga, sgb,
                     rows_p, ssem_p):
            for g in range(EB // 16):
                sl = pl.ds(g * 16, 16)
                pltpu.make_async_copy(zall.at[pl.ds(0, 16)],
                                      zsrc_p.at[sl], sga).wait()
                pltpu.make_async_copy(zall.at[pl.ds(0, 16)],
                                      zdst_p.at[sl], sgb).wait()

            def _edge(r2, carry3):
                for dr in range(1):
                    r = r2 + dr
                    rfull = jnp.full((16,), r, jnp.int32)
                    av = [plsc.load_gather(zsrc_p, [rfull, j * 16 + iota16])
                          for j in range(HID // 16)]
                    bv = [plsc.load_gather(zdst_p, [rfull, j * 16 + iota16])
                          for j in range(HID // 16)]
                    dvec = jnp.zeros((16,), jnp.float32)
                    for k in range(NUM_LATENT):
                        p = (av[2 * k] * bv[2 * k]
                             + av[2 * k + 1] * bv[2 * k + 1])
                        s = jnp.sum(p)
                        e = jnp.maximum(s, s * 0.2)
                        eev = jnp.exp(jnp.full((16,), e, jnp.float32))
                        plsc.store_scatter(rows_p, [rfull, k * DK + iota16],
                                           av[2 * k] * eev)
                        plsc.store_scatter(rows_p,
                                           [rfull, k * DK + 16 + iota16],
                                           av[2 * k + 1] * eev)
                        dvec = dvec + eev * latm[k]
                    plsc.store_scatter(rows_p, [rfull, 128 + iota16], dvec)
                return carry3
            lax.fori_loop(0, EB, _edge, 0)

            pltpu.async_copy(rows_p, acc.at[dslot_p], ssem_p, add=True)

        p0 = (gsrc_v, gdst_v, dslot_v, zsrc, zdst, sem3, sem4, rows, sem7)
        p1 = (gsrc2, gdst2, dslot2, zsrc2, zdst2, sem5, sem6, rows2, sem8)

        @pl.when(nb > 0)
        def _():
            _fire(0, *p0)

        def _pairs(i, carry2):
            b0 = 2 * i
            b1 = b0 + 1

            @pl.when(b1 < nb)
            def _():
                _fire(b1, *p1)

            @pl.when(b0 < nb)
            def _():
                _process(b0, *p0)

            @pl.when(b1 + 1 < nb)
            def _():
                _fire(b1 + 1, *p0)

            @pl.when(b1 < nb)
            def _():
                _process(b1, *p1)
            return carry2
        lax.fori_loop(0, (nb + 1) // 2, _pairs, 0)

        @pl.when(((nb >= 1) & ((nb - 1) % 2 == 0)) | (nb >= 2))
        def _():
            pltpu.make_async_copy(hbm_dummy, rows, sem7).wait()

        @pl.when((nb >= 2) | ((nb >= 1) & ((nb - 1) % 2 == 1)))
        def _():
            pltpu.make_async_copy(hbm_dummy, rows2, sem8).wait()
        return carry
    lax.fori_loop(0, EPT // SB, _super, 0)

    plsc.subcore_barrier()

    for ch in range(TPT // 32):
        def _tslot(g, carry, ch=ch):
            tv = tgt_all[pl.ds(tb0 + ch * 32 + g * 16, 16)]
            tslot_v[pl.ds(g * 16, 16)] = plsc.load_gather(m_tab, [tv])
            return carry
        lax.fori_loop(0, 2, _tslot, 0)

        pltpu.async_copy(acc.at[tslot_v], trow, sem1).wait()

        def _nrm(i, carry):
            ifull = jnp.full((16,), i, jnp.int32)
            for k in range(NUM_LATENT):
                dk = plsc.load_gather(
                    trow, [ifull, jnp.full((16,), 128 + k, jnp.int32)])
                dk = dk + 1e-9
                for j2 in range(2):
                    off = k * DK + j2 * 16
                    v = plsc.load_gather(trow, [ifull, off + iota16]) / dk
                    plsc.store_scatter(mrow, [ifull, off + iota16], v)
            return carry
        lax.fori_loop(0, 32, _nrm, 0)

        pltpu.sync_copy(
            mrow, mout.at[c, pl.ds(pl.multiple_of(tb0 + ch * 32, 8), 32)])


def _sc_aggregate(zall, nidx, esrc, edst, tgt):
    mesh = plsc.VectorSubcoreMesh(core_axis_name="c", subcore_axis_name="s")
    return pl.kernel(
        _sc_body,
        out_type=jax.ShapeDtypeStruct((2, T, HID), jnp.float32),
        mesh=mesh,
        compiler_params=pltpu.CompilerParams(use_tc_tiling_on_sc=False,
                                             needs_layout_passes=False),
        scratch_types=[
            pltpu.VMEM((N_SUB,), jnp.int32),
            pltpu.VMEM((N_SUB,), jnp.int32),
            pltpu.VMEM((T,), jnp.int32),
            pltpu.VMEM((SB,), jnp.int32),
            pltpu.VMEM((SB,), jnp.int32),
            pltpu.VMEM((EB,), jnp.int32),
            pltpu.VMEM((EB,), jnp.int32),
            pltpu.VMEM((EB,), jnp.int32),
            pltpu.VMEM((32,), jnp.int32),
            pltpu.VMEM((EB, HID), jnp.float32),
            pltpu.VMEM((EB, HID), jnp.float32),
            pltpu.VMEM((EB, ACCW), jnp.float32),
            pltpu.VMEM((32, ACCW), jnp.float32),
            pltpu.VMEM((32, HID), jnp.float32),
            pltpu.VMEM((KCAP,), jnp.int32),
            pltpu.VMEM((KCAP,), jnp.int32),
            pltpu.VMEM((KCAP,), jnp.int32),
            pltpu.VMEM((EB,), jnp.int32),
            pltpu.VMEM((EB,), jnp.int32),
            pltpu.VMEM((EB,), jnp.int32),
            pltpu.VMEM((EB, HID), jnp.float32),
            pltpu.VMEM((EB, HID), jnp.float32),
            pltpu.VMEM((EB, ACCW), jnp.float32),
            pltpu.HBM((EB, ACCW), jnp.float32),
            pltpu.VMEM_SHARED((NACC, ACCW), jnp.float32),
            pltpu.VMEM_SHARED((NACC, HID), jnp.float32),
        ] + [pltpu.SemaphoreType.DMA] * 8,
    )(zall, nidx, esrc, edst, tgt)


def _out_body(m_ref, w_ref, b_ref, o_ref):
    o = jnp.dot(m_ref[0], w_ref[0], preferred_element_type=jnp.float32)
    o_ref[0] = o + b_ref[0]


def _project_out(mout, w, b):
    return pl.pallas_call(
        _out_body,
        grid=(2,),
        in_specs=[
            pl.BlockSpec((1, T, HID), lambda j: (j, 0, 0)),
            pl.BlockSpec((1, HID, OUT_DIM), lambda j: (j, 0, 0)),
            pl.BlockSpec((1, 1, OUT_DIM), lambda j: (j, 0, 0)),
        ],
        out_specs=pl.BlockSpec((1, T, OUT_DIM), lambda j: (j, 0, 0)),
        out_shape=jax.ShapeDtypeStruct((2, T, OUT_DIM), jnp.float32),
    )(mout, w, b)


def kernel(feat0, feat1, type_mask, node_idx_gene, node_idx_dis,
           edge_index_gene, edge_index_dis, target_idx_gene, target_idx_dis,
           fc_type_W, fc_type_b, gene_Wf, gene_bf, gene_fc1_W, gene_fc1_b,
           gene_fc2_W, gene_fcout_W, gene_fcout_b, dis_Wf, dis_bf, dis_fc1_W,
           dis_fc1_b, dis_fc2_W, dis_fcout_W, dis_fcout_b):
    feats = jnp.concatenate([feat0, feat1], axis=0)
    wz = jnp.stack([
        jnp.transpose(gene_Wf, (1, 0, 2)).reshape(HID, HID),
        jnp.transpose(dis_Wf, (1, 0, 2)).reshape(HID, HID),
    ])
    bz = jnp.stack([gene_bf.reshape(1, HID), dis_bf.reshape(1, HID)])

    zall = _make_ztables(feats, fc_type_W, fc_type_b.reshape(2, 1, HID), wz, bz)
    zflat = zall.reshape(2 * N_TOTAL, HID)

    nidx = jnp.concatenate([node_idx_gene, node_idx_dis])
    esrc = jnp.concatenate([edge_index_gene[0], edge_index_dis[0]])
    edst = jnp.concatenate([edge_index_gene[1], edge_index_dis[1]])
    tgt = jnp.concatenate([target_idx_gene, target_idx_dis])

    mout = _sc_aggregate(zflat, nidx, esrc, edst, tgt)

    wout = jnp.stack([gene_fcout_W, dis_fcout_W])
    bout = jnp.stack([gene_fcout_b.reshape(1, OUT_DIM),
                      dis_fcout_b.reshape(1, OUT_DIM)])
    logits = _project_out(mout, wout, bout)
    return (logits[0], logits[1])
